# baseline (speedup 1.0000x reference)
import functools

import jax
import jax.numpy as jnp
from jax.experimental import pallas as pl
from jax.experimental.pallas import tpu as pltpu
from jax.experimental.pallas import tpu_sc as plsc

_mesh = plsc.ScalarSubcoreMesh(axis_name="c", num_cores=1)

_ROW_MAP = ((0, 0), (6, 1), (1, 2))


@functools.partial(
    pl.kernel,
    mesh=_mesh,
    out_type=jax.ShapeDtypeStruct((2, 3, 128), jnp.float32),
)
def _gather_rows(slab_hbm, out_hbm):
    pltpu.sync_copy(slab_hbm.at[0, pl.ds(2, 3)], out_hbm.at[0])
    for src_j, dst_b in _ROW_MAP:
        pltpu.sync_copy(
            slab_hbm.at[1, pl.ds(src_j, 1)], out_hbm.at[1, pl.ds(dst_b, 1)]
        )


def kernel(x):
    slab = jax.lax.slice(x, (0, 0, 0), (2, 8, 128))
    return _gather_rows(slab)


# speedup vs baseline: 5.6642x; 5.6642x over previous
import jax
import jax.numpy as jnp
from jax.experimental import pallas as pl
from jax.experimental.pallas import tpu as pltpu


def _body(slab_hbm, out_hbm, sem):
    copies = [
        pltpu.make_async_copy(slab_hbm.at[0, pl.ds(2, 3)], out_hbm.at[0], sem),
        pltpu.make_async_copy(
            slab_hbm.at[1, pl.ds(0, 1)], out_hbm.at[1, pl.ds(0, 1)], sem
        ),
        pltpu.make_async_copy(
            slab_hbm.at[1, pl.ds(6, 1)], out_hbm.at[1, pl.ds(1, 1)], sem
        ),
        pltpu.make_async_copy(
            slab_hbm.at[1, pl.ds(1, 1)], out_hbm.at[1, pl.ds(2, 1)], sem
        ),
    ]
    for c in copies:
        c.start()
    for c in copies:
        c.wait()


def kernel(x):
    slab = jax.lax.slice(x, (0, 0, 0), (2, 8, 128))
    return pl.pallas_call(
        _body,
        in_specs=[pl.BlockSpec(memory_space=pl.ANY)],
        out_specs=pl.BlockSpec(memory_space=pl.ANY),
        out_shape=jax.ShapeDtypeStruct((2, 3, 128), jnp.float32),
        scratch_shapes=[pltpu.SemaphoreType.DMA],
    )(slab)
